# baseline (speedup 1.0000x reference)
import math

import jax
import jax.numpy as jnp
from jax import lax
from jax.experimental import pallas as pl
from jax.experimental.pallas import tpu as pltpu
from jax.experimental.pallas import tpu_sc as plsc

VOCAB = 1000000
D = 64
SCALE = math.sqrt(D)

NC, NS, L = 2, 16, 16
NW = NC * NS

B = 4096 * 200
B_W = B // NW
C = 512
N_CHUNKS = B_W // C


def _body(x_hbm, table_hbm, out_hbm, idx_v, rows_v, sem):
    wid = lax.axis_index("s") * NC + lax.axis_index("c")
    base = wid * B_W

    def chunk(g, carry):
        off = base + g * C
        pltpu.sync_copy(x_hbm.at[pl.ds(off, C)], idx_v)
        pltpu.async_copy(table_hbm.at[idx_v], rows_v, sem).wait()

        def srow(j, carry2):
            for l in range(D // L):
                sl = pl.ds(l * L, L)
                rows_v[j, sl] = rows_v[j, sl] * SCALE
            return carry2

        lax.fori_loop(0, C, srow, 0)
        pltpu.sync_copy(rows_v, out_hbm.at[pl.ds(off, C)])
        return carry

    lax.fori_loop(0, N_CHUNKS, chunk, 0)


@jax.jit
def kernel(x, table):
    x_flat = x.reshape(-1).astype(jnp.int32)
    mesh = plsc.VectorSubcoreMesh(
        core_axis_name="c", subcore_axis_name="s", num_cores=NC, num_subcores=NS
    )
    out = pl.kernel(
        _body,
        out_type=jax.ShapeDtypeStruct((B, D), jnp.float32),
        mesh=mesh,
        scratch_types=[
            pltpu.VMEM((C,), jnp.int32),
            pltpu.VMEM((C, D), jnp.float32),
            pltpu.SemaphoreType.DMA,
        ],
        compiler_params=pltpu.CompilerParams(use_tc_tiling_on_sc=False),
    )(x_flat, table)
    return out.reshape(x.shape[0], x.shape[1], D)


# speedup vs baseline: 1.1359x; 1.1359x over previous
import math

import jax
import jax.numpy as jnp
from jax import lax
from jax.experimental import pallas as pl
from jax.experimental.pallas import tpu as pltpu
from jax.experimental.pallas import tpu_sc as plsc

VOCAB = 1000000
D = 64
SCALE = math.sqrt(D)

NC, NS, L = 2, 16, 16
NW = NC * NS

B = 4096 * 200
B_W = B // NW
C = 256
N_CHUNKS = B_W // C
NBUF = 4
LOOKAHEAD = 2


def _body(x_hbm, table_hbm, out_hbm,
          idx_all, rows0, rows1, rows2, rows3,
          si0, si1, si2, si3, so0, so1, so2, so3):
    rows = (rows0, rows1, rows2, rows3)
    sem_in = (si0, si1, si2, si3)
    sem_out = (so0, so1, so2, so3)

    wid = lax.axis_index("s") * NC + lax.axis_index("c")
    base = wid * B_W
    pltpu.sync_copy(x_hbm.at[pl.ds(base, B_W)], idx_all)

    def gather_start(g, b):
        pltpu.async_copy(table_hbm.at[idx_all.at[pl.ds(g * C, C)]],
                         rows[b], sem_in[b])

    def gather_wait(g, b):
        pltpu.make_async_copy(table_hbm.at[idx_all.at[pl.ds(g * C, C)]],
                              rows[b], sem_in[b]).wait()

    def store_start(g, b):
        pltpu.async_copy(rows[b], out_hbm.at[pl.ds(base + g * C, C)],
                         sem_out[b])

    def store_wait(b):
        pltpu.make_async_copy(rows[b], out_hbm.at[pl.ds(base, C)],
                              sem_out[b]).wait()

    for b in range(LOOKAHEAD):
        gather_start(b, b)

    def outer(t, carry):
        g0 = t * NBUF
        for b in range(NBUF):
            g = g0 + b
            gather_wait(g, b)

            rb = rows[b]

            @plsc.parallel_loop(0, C, step=1, unroll=2)
            def _(j):
                for l in range(D // L):
                    sl = pl.ds(l * L, L)
                    rb[j, sl] = rb[j, sl] * SCALE

            store_start(g, b)

            gnext = g + LOOKAHEAD
            bnext = (b + LOOKAHEAD) % NBUF

            @pl.when(jnp.logical_and(g >= NBUF - LOOKAHEAD,
                                     gnext < N_CHUNKS))
            def _():
                store_wait(bnext)

            @pl.when(gnext < N_CHUNKS)
            def _():
                gather_start(gnext, bnext)

        return carry

    lax.fori_loop(0, N_CHUNKS // NBUF, outer, 0)

    for b in range(NBUF):
        store_wait(b)


@jax.jit
def kernel(x, table):
    x_flat = x.reshape(-1).astype(jnp.int32)
    mesh = plsc.VectorSubcoreMesh(
        core_axis_name="c", subcore_axis_name="s", num_cores=NC, num_subcores=NS
    )
    out = pl.kernel(
        _body,
        out_type=jax.ShapeDtypeStruct((B, D), jnp.float32),
        mesh=mesh,
        scratch_types=(
            [pltpu.VMEM((B_W,), jnp.int32)]
            + [pltpu.VMEM((C, D), jnp.float32) for _ in range(NBUF)]
            + [pltpu.SemaphoreType.DMA for _ in range(2 * NBUF)]
        ),
        compiler_params=pltpu.CompilerParams(use_tc_tiling_on_sc=False),
    )(x_flat, table)
    return out.reshape(x.shape[0], x.shape[1], D)


# speedup vs baseline: 1.6486x; 1.4514x over previous
import math

import jax
import jax.numpy as jnp
from jax import lax
from jax.experimental import pallas as pl
from jax.experimental.pallas import tpu as pltpu
from jax.experimental.pallas import tpu_sc as plsc

VOCAB = 1000000
D = 64
SCALE = math.sqrt(D)

NC, NS, L = 2, 16, 16
J = 200
I = 4096
IPT = I // NS
DPC = D // NC
NV = J * IPT
JC = 50
NCH = J // JC


def _body(xT_hbm, tableT_hbm, out_hbm, idx_v, gath, col_a,
          sem_stage, sem_i, sem_g, sem_o):
    c = lax.axis_index("c")
    s = lax.axis_index("s")
    i0 = s * IPT

    def idx_fire(j, cy):
        pltpu.async_copy(xT_hbm.at[j, pl.ds(i0, IPT)],
                         idx_v.at[pl.ds(j * IPT, IPT)], sem_i)
        return cy

    lax.fori_loop(0, J, idx_fire, 0)

    def idx_drain(j, cy):
        pltpu.make_async_copy(xT_hbm.at[j, pl.ds(i0, IPT)],
                              idx_v.at[pl.ds(j * IPT, IPT)], sem_i).wait()
        return cy

    lax.fori_loop(0, J, idx_drain, 0)

    def stage_start(dd, col):
        @pl.when(s == 0)
        def _():
            pltpu.async_copy(tableT_hbm.at[c * DPC + dd], col, sem_stage)

    def stage_wait(dd, col):
        @pl.when(s == 0)
        def _():
            pltpu.make_async_copy(
                tableT_hbm.at[c * DPC + dd], col, sem_stage).wait()


    def process_d(dd, carry):
        col_cur = col_a
        dglob = c * DPC + dd
        stage_start(dd, col_cur)
        stage_wait(dd, col_cur)
        plsc.subcore_barrier()

        def chunk(ch, cy2):
            j0 = ch * JC

            def g_fire(j, cy):
                pltpu.async_copy(col_cur.at[idx_v.at[pl.ds((j0 + j) * IPT, IPT)]],
                                 gath.at[pl.ds(j * IPT, IPT)], sem_g)
                return cy

            lax.fori_loop(0, JC, g_fire, 0)

            def g_drain(j, cy):
                pltpu.make_async_copy(
                    col_cur.at[idx_v.at[pl.ds((j0 + j) * IPT, IPT)]],
                    gath.at[pl.ds(j * IPT, IPT)], sem_g).wait()
                return cy

            lax.fori_loop(0, JC, g_drain, 0)

            @plsc.parallel_loop(0, JC * IPT // L, step=1, unroll=4)
            def _(v):
                sl = pl.ds(v * L, L)
                gath[sl] = gath[sl] * SCALE

            def o_fire(j, cy):
                pltpu.async_copy(gath.at[pl.ds(j * IPT, IPT)],
                                 out_hbm.at[j0 + j, dglob, pl.ds(i0, IPT)], sem_o)
                return cy

            lax.fori_loop(0, JC, o_fire, 0)

            def o_drain(j, cy):
                pltpu.make_async_copy(
                    gath.at[pl.ds(j * IPT, IPT)],
                    out_hbm.at[j0 + j, dglob, pl.ds(i0, IPT)], sem_o).wait()
                return cy

            lax.fori_loop(0, JC, o_drain, 0)
            return cy2

        lax.fori_loop(0, NCH, chunk, 0)

        plsc.subcore_barrier()
        return carry

    lax.fori_loop(0, DPC, process_d, 0)


@jax.jit
def kernel(x, table):
    xT = x.T.astype(jnp.int32)
    tableT = table.T
    mesh = plsc.VectorSubcoreMesh(
        core_axis_name="c", subcore_axis_name="s", num_cores=NC, num_subcores=NS
    )
    out_t = pl.kernel(
        _body,
        out_type=jax.ShapeDtypeStruct((J, D, I), jnp.float32),
        mesh=mesh,
        scratch_types=[
            pltpu.VMEM((NV,), jnp.int32),
            pltpu.VMEM((JC * IPT,), jnp.float32),
            pltpu.VMEM_SHARED((VOCAB,), jnp.float32),
            pltpu.SemaphoreType.DMA,
            pltpu.SemaphoreType.DMA,
            pltpu.SemaphoreType.DMA,
            pltpu.SemaphoreType.DMA,
        ],
        compiler_params=pltpu.CompilerParams(use_tc_tiling_on_sc=True),
    )(xT, tableT)
    return out_t.transpose(2, 0, 1)


# speedup vs baseline: 1.9012x; 1.1533x over previous
import math

import jax
import jax.numpy as jnp
from jax import lax
from jax.experimental import pallas as pl
from jax.experimental.pallas import tpu as pltpu
from jax.experimental.pallas import tpu_sc as plsc

VOCAB = 1000000
D = 64
SCALE = math.sqrt(D)

NC, NS, L = 2, 16, 16
J = 200
I = 4096
IPT = I // NS
DPC = D // NC
NV = J * IPT
JC = 25
NCH = J // JC
CV = JC * IPT


def _body(xT_hbm, tableT_hbm, out_hbm, idx_v, gath_a, gath_b, col,
          sem_stage, sem_i, sem_g, sem_o):
    c = lax.axis_index("c")
    s = lax.axis_index("s")
    i0 = s * IPT

    def idx_fire(j, cy):
        pltpu.async_copy(xT_hbm.at[j, pl.ds(i0, IPT)],
                         idx_v.at[pl.ds(j * IPT, IPT)], sem_i)
        return cy

    lax.fori_loop(0, J, idx_fire, 0)

    def idx_drain(j, cy):
        pltpu.make_async_copy(xT_hbm.at[j, pl.ds(i0, IPT)],
                              idx_v.at[pl.ds(j * IPT, IPT)], sem_i).wait()
        return cy

    lax.fori_loop(0, J, idx_drain, 0)

    def process_d(dd, carry):
        dglob = c * DPC + dd

        @pl.when(s == 0)
        def _():
            pltpu.async_copy(tableT_hbm.at[dglob], col, sem_stage)
            pltpu.make_async_copy(tableT_hbm.at[dglob], col, sem_stage).wait()

        plsc.subcore_barrier()

        def g_fire(ch, buf):
            pltpu.async_copy(col.at[idx_v.at[pl.ds(ch * CV, CV)]], buf, sem_g)

        def g_drain(ch, buf):
            pltpu.make_async_copy(col.at[idx_v.at[pl.ds(ch * CV, CV)]],
                                  buf, sem_g).wait()

        def scale(buf):
            @plsc.parallel_loop(0, CV // L, step=1, unroll=4)
            def _(v):
                sl = pl.ds(v * L, L)
                buf[sl] = buf[sl] * SCALE

        def o_fire(ch, buf):
            j0 = ch * JC

            def one(j, cy):
                pltpu.async_copy(buf.at[pl.ds(j * IPT, IPT)],
                                 out_hbm.at[j0 + j, dglob, pl.ds(i0, IPT)],
                                 sem_o)
                return cy

            lax.fori_loop(0, JC, one, 0)

        def o_drain(ch, buf):
            j0 = ch * JC

            def one(j, cy):
                pltpu.make_async_copy(
                    buf.at[pl.ds(j * IPT, IPT)],
                    out_hbm.at[j0 + j, dglob, pl.ds(i0, IPT)], sem_o).wait()
                return cy

            lax.fori_loop(0, JC, one, 0)

        g_fire(0, gath_a)

        def pair(t, cy):
            ca = 2 * t
            cb = 2 * t + 1

            g_drain(ca, gath_a)

            @pl.when(t >= 1)
            def _():
                o_drain(cb - 2, gath_b)

            g_fire(cb, gath_b)
            scale(gath_a)
            o_fire(ca, gath_a)

            g_drain(cb, gath_b)
            o_drain(ca, gath_a)

            @pl.when(cb + 1 < NCH)
            def _():
                g_fire(cb + 1, gath_a)

            scale(gath_b)
            o_fire(cb, gath_b)
            return cy

        lax.fori_loop(0, NCH // 2, pair, 0)
        o_drain(NCH - 1, gath_b)

        plsc.subcore_barrier()
        return carry

    lax.fori_loop(0, DPC, process_d, 0)


@jax.jit
def kernel(x, table):
    xT = x.T.astype(jnp.int32)
    tableT = table.T
    mesh = plsc.VectorSubcoreMesh(
        core_axis_name="c", subcore_axis_name="s", num_cores=NC, num_subcores=NS
    )
    out_t = pl.kernel(
        _body,
        out_type=jax.ShapeDtypeStruct((J, D, I), jnp.float32),
        mesh=mesh,
        scratch_types=[
            pltpu.VMEM((NV,), jnp.int32),
            pltpu.VMEM((CV,), jnp.float32),
            pltpu.VMEM((CV,), jnp.float32),
            pltpu.VMEM_SHARED((VOCAB,), jnp.float32),
            pltpu.SemaphoreType.DMA,
            pltpu.SemaphoreType.DMA,
            pltpu.SemaphoreType.DMA,
            pltpu.SemaphoreType.DMA,
        ],
        compiler_params=pltpu.CompilerParams(use_tc_tiling_on_sc=True),
    )(xT, tableT)
    return out_t.transpose(2, 0, 1)


# speedup vs baseline: 1.9581x; 1.0299x over previous
import math

import jax
import jax.numpy as jnp
from jax import lax
from jax.experimental import pallas as pl
from jax.experimental.pallas import tpu as pltpu
from jax.experimental.pallas import tpu_sc as plsc

VOCAB = 1000000
D = 64
SCALE = math.sqrt(D)

NC, NS, L = 2, 16, 16
J = 200
I = 4096
IPT = I // NS
DPC = D // NC
NV = J * IPT
JC = 25
NCH = J // JC
CV = JC * IPT


def _body(xT_hbm, tableT_hbm, out_hbm, idx_v, gath_a, gath_b, col,
          sem_stage, sem_i, sem_g, sem_o):
    c = lax.axis_index("c")
    s = lax.axis_index("s")
    i0 = s * IPT

    def idx_fire(j, cy):
        pltpu.async_copy(xT_hbm.at[j, pl.ds(i0, IPT)],
                         idx_v.at[pl.ds(j * IPT, IPT)], sem_i)
        return cy

    lax.fori_loop(0, J, idx_fire, 0)

    def idx_drain(j, cy):
        pltpu.make_async_copy(xT_hbm.at[j, pl.ds(i0, IPT)],
                              idx_v.at[pl.ds(j * IPT, IPT)], sem_i).wait()
        return cy

    lax.fori_loop(0, J, idx_drain, 0)

    def stage_fire(dd):
        @pl.when(s == 0)
        def _():
            pltpu.async_copy(tableT_hbm.at[c * DPC + dd], col, sem_stage)

    def stage_wait(dd):
        @pl.when(s == 0)
        def _():
            pltpu.make_async_copy(tableT_hbm.at[c * DPC + dd], col,
                                  sem_stage).wait()

    stage_fire(0)
    stage_wait(0)
    plsc.subcore_barrier()

    def process_d(dd, carry):
        dglob = c * DPC + dd

        plsc.subcore_barrier()

        def g_fire(ch, buf):
            pltpu.async_copy(col.at[idx_v.at[pl.ds(ch * CV, CV)]], buf, sem_g)

        def g_drain(ch, buf):
            pltpu.make_async_copy(col.at[idx_v.at[pl.ds(ch * CV, CV)]],
                                  buf, sem_g).wait()

        def scale(buf):
            @plsc.parallel_loop(0, CV // L, step=1, unroll=4)
            def _(v):
                sl = pl.ds(v * L, L)
                buf[sl] = buf[sl] * SCALE

        def o_fire(ch, buf):
            j0 = ch * JC

            def one(j, cy):
                pltpu.async_copy(buf.at[pl.ds(j * IPT, IPT)],
                                 out_hbm.at[j0 + j, dglob, pl.ds(i0, IPT)],
                                 sem_o)
                return cy

            lax.fori_loop(0, JC, one, 0)

        def o_drain(ch, buf):
            j0 = ch * JC

            def one(j, cy):
                pltpu.make_async_copy(
                    buf.at[pl.ds(j * IPT, IPT)],
                    out_hbm.at[j0 + j, dglob, pl.ds(i0, IPT)], sem_o).wait()
                return cy

            lax.fori_loop(0, JC, one, 0)

        g_fire(0, gath_a)

        def pair(t, cy):
            ca = 2 * t
            cb = 2 * t + 1

            g_drain(ca, gath_a)

            @pl.when(t >= 1)
            def _():
                o_drain(cb - 2, gath_b)

            g_fire(cb, gath_b)
            scale(gath_a)
            o_fire(ca, gath_a)

            g_drain(cb, gath_b)
            o_drain(ca, gath_a)

            @pl.when(cb + 1 < NCH)
            def _():
                g_fire(cb + 1, gath_a)

            scale(gath_b)
            o_fire(cb, gath_b)
            return cy

        lax.fori_loop(0, NCH // 2 - 1, pair, 0)

        ca, cb = NCH - 2, NCH - 1
        g_drain(ca, gath_a)
        o_drain(cb - 2, gath_b)
        g_fire(cb, gath_b)
        scale(gath_a)
        o_fire(ca, gath_a)
        g_drain(cb, gath_b)
        plsc.subcore_barrier()

        @pl.when(dd + 1 < DPC)
        def _():
            stage_fire(dd + 1)

        o_drain(ca, gath_a)
        scale(gath_b)
        o_fire(cb, gath_b)
        o_drain(cb, gath_b)

        @pl.when(dd + 1 < DPC)
        def _():
            stage_wait(dd + 1)

        plsc.subcore_barrier()
        return carry

    lax.fori_loop(0, DPC, process_d, 0)


@jax.jit
def kernel(x, table):
    xT = x.T.astype(jnp.int32)
    tableT = table.T
    mesh = plsc.VectorSubcoreMesh(
        core_axis_name="c", subcore_axis_name="s", num_cores=NC, num_subcores=NS
    )
    out_t = pl.kernel(
        _body,
        out_type=jax.ShapeDtypeStruct((J, D, I), jnp.float32),
        mesh=mesh,
        scratch_types=[
            pltpu.VMEM((NV,), jnp.int32),
            pltpu.VMEM((CV,), jnp.float32),
            pltpu.VMEM((CV,), jnp.float32),
            pltpu.VMEM_SHARED((VOCAB,), jnp.float32),
            pltpu.SemaphoreType.DMA,
            pltpu.SemaphoreType.DMA,
            pltpu.SemaphoreType.DMA,
            pltpu.SemaphoreType.DMA,
        ],
        compiler_params=pltpu.CompilerParams(use_tc_tiling_on_sc=True),
    )(xT, tableT)
    return out_t.transpose(2, 0, 1)
